# baseline (speedup 1.0000x reference)
import functools

import jax
import jax.numpy as jnp
from jax import lax
from jax.experimental import pallas as pl
from jax.experimental.pallas import tpu as pltpu
from jax.experimental.pallas import tpu_sc as plsc

_N = 10000
_NP = 10240
_E = 320000
_EP = 327680
_H = 128
_LAYERS = 7
_EPS = 1e-7
_CH = 64
_ECHUNKS = _EP // _CH


@functools.lru_cache(maxsize=None)
def _sc_info():
    info = plsc.get_sparse_core_info()
    return info.num_cores, info.num_subcores


@functools.lru_cache(maxsize=None)
def _build_sc_encode():
    nc, ns = _sc_info()
    ntiles = nc * ns
    nchunk = _ECHUNKS // ntiles
    rows_per_tile = _NP // ntiles
    xrows = _NP // 40
    crows = 512 // ntiles
    mesh = plsc.VectorSubcoreMesh(core_axis_name="c", subcore_axis_name="s")

    def body(x9r, a0, a1, a2, a3, a4, a5, a6, a7, a8, bcat, ea0, ea1, ea2,
             henc, ctab, eidx,
             idxf, abuf, hacc, bb, ctb, e0v, e1v, e2v, sem):
        aembs = [a0, a1, a2, a3, a4, a5, a6, a7, a8]
        cid = lax.axis_index("c")
        sid = lax.axis_index("s")
        wid = cid * ns + sid

        pltpu.sync_copy(bcat, bb)

        @pl.loop(0, crows)
        def _(t):
            i = wid * crows + t
            r0 = i & 7
            r1 = 8 + ((i >> 3) & 7)
            r2 = 16 + ((i >> 6) & 7)
            for c in range(_H // 16):
                sl = pl.ds(c * 16, 16)
                ctb[t, sl] = bb[r0, sl] + bb[r1, sl] + bb[r2, sl]

        pltpu.sync_copy(ctb, ctab.at[pl.ds(wid * crows, crows)])

        ebase = wid * nchunk
        pltpu.sync_copy(ea0.at[pl.ds(ebase, nchunk)], e0v)
        pltpu.sync_copy(ea1.at[pl.ds(ebase, nchunk)], e1v)
        pltpu.sync_copy(ea2.at[pl.ds(ebase, nchunk)], e2v)

        @pl.loop(0, nchunk)
        def _(r):
            for c in range(_CH // 16):
                sl = pl.ds(c * 16, 16)
                e0v[r, sl] = e0v[r, sl] + (e1v[r, sl] << 3) + (e2v[r, sl] << 6)

        pltpu.sync_copy(e0v, eidx.at[pl.ds(ebase, nchunk)])

        for f in range(9):
            pltpu.sync_copy(x9r.at[pl.ds(f * xrows + wid * 8, 8)], idxf)
            for k in range(8):
                if f == 0:
                    pltpu.async_copy(
                        aembs[f].at[idxf.at[k]],
                        hacc.at[pl.ds(k * 40, 40)], sem).wait()
                else:
                    pltpu.async_copy(aembs[f].at[idxf.at[k]], abuf, sem).wait()

                    @pl.loop(0, 40)
                    def _(r):
                        for c in range(_H // 16):
                            sl = pl.ds(c * 16, 16)
                            hacc[k * 40 + r, sl] = hacc[k * 40 + r, sl] + abuf[r, sl]

        pltpu.sync_copy(hacc, henc.at[pl.ds(wid * rows_per_tile, rows_per_tile)])

    return pl.kernel(
        body,
        out_type=(
            jax.ShapeDtypeStruct((_NP, _H), jnp.float32),
            jax.ShapeDtypeStruct((512, _H), jnp.float32),
            jax.ShapeDtypeStruct((_ECHUNKS, _CH), jnp.int32),
        ),
        mesh=mesh,
        scratch_types=(
            pltpu.VMEM((8, 40), jnp.int32),
            pltpu.VMEM((40, _H), jnp.float32),
            pltpu.VMEM((rows_per_tile, _H), jnp.float32),
            pltpu.VMEM((24, _H), jnp.float32),
            pltpu.VMEM((crows, _H), jnp.float32),
            pltpu.VMEM((nchunk, _CH), jnp.int32),
            pltpu.VMEM((nchunk, _CH), jnp.int32),
            pltpu.VMEM((nchunk, _CH), jnp.int32),
            pltpu.SemaphoreType.DMA,
        ),
    )


@functools.lru_cache(maxsize=None)
def _build_sc_msg():
    nc, ns = _sc_info()
    ntiles = nc * ns
    nchunk = _ECHUNKS // ntiles
    ngroup = nchunk // 8
    mrows = _NP // ns
    mesh = plsc.VectorSubcoreMesh(core_axis_name="c", subcore_axis_name="s")

    def body(hcur, srcr, eidxr, dstr, ctab,
             mout,
             sv, ev, dv, h0, h1, e0, e1, msh,
             gh0, gh1, ge0, ge1, ss0, ss1):
        cid = lax.axis_index("c")
        sid = lax.axis_index("s")
        wid = cid * ns + sid
        base = wid * nchunk
        hb = (h0, h1)
        eb = (e0, e1)
        gh = (gh0, gh1)
        ge = (ge0, ge1)
        ss = (ss0, ss1)

        def stage(grp, half):
            gb = base + grp * 8
            hs = pl.ds(half * 8, 8)
            pltpu.sync_copy(srcr.at[pl.ds(gb, 8)], sv.at[hs])
            pltpu.sync_copy(eidxr.at[pl.ds(gb, 8)], ev.at[hs])
            pltpu.sync_copy(dstr.at[pl.ds(gb, 8)], dv.at[hs])

        stage(0, 0)
        for buf in (h0, h1):
            @pl.loop(0, _CH)
            def _(r):
                for c in range(_H // 16):
                    buf[r, pl.ds(c * 16, 16)] = jnp.zeros((16,), jnp.float32)

        mbase = sid * mrows
        for k in range(mrows // _CH):
            pltpu.sync_copy(h0, msh.at[pl.ds(mbase + k * _CH, _CH)])
        plsc.subcore_barrier()

        pltpu.async_copy(h1, msh.at[dv.at[0]], ss1, add=True)
        pltpu.async_copy(hcur.at[sv.at[0]], h0, gh0)
        pltpu.async_copy(ctab.at[ev.at[0]], e0, ge0)

        @pl.loop(0, ngroup)
        def _(g):
            par = g & 1
            par2 = (g + 1) & 1
            for j in range(8):
                b = j & 1
                c = g * 8 + j
                pltpu.make_async_copy(hcur.at[sv.at[0]], hb[b], gh[b]).wait()
                pltpu.make_async_copy(ctab.at[ev.at[0]], eb[b], ge[b]).wait()

                @pl.when(c < nchunk - 1)
                def _():
                    pltpu.make_async_copy(
                        hb[1 - b], msh.at[dv.at[0]], ss[1 - b]).wait()
                    if j < 7:
                        nrow = par * 8 + j + 1
                    else:
                        nrow = par2 * 8
                    pltpu.async_copy(hcur.at[sv.at[nrow]], hb[1 - b], gh[1 - b])
                    pltpu.async_copy(ctab.at[ev.at[nrow]], eb[1 - b], ge[1 - b])

                if j == 2:
                    @pl.when(g < ngroup - 1)
                    def _():
                        stage(g + 1, par2)

                @pl.loop(0, _CH)
                def _(r):
                    for cc in range(_H // 16):
                        sl = pl.ds(cc * 16, 16)
                        v = hb[b][r, sl] + eb[b][r, sl]
                        hb[b][r, sl] = jnp.maximum(v, 0.0) + _EPS

                pltpu.async_copy(hb[b], msh.at[dv.at[par * 8 + j]], ss[b],
                                 add=True)

        pltpu.make_async_copy(h0, msh.at[dv.at[0]], ss0).wait()
        pltpu.make_async_copy(h1, msh.at[dv.at[0]], ss1).wait()
        plsc.subcore_barrier()
        pltpu.sync_copy(msh.at[pl.ds(mbase, mrows)],
                        mout.at[cid, pl.ds(mbase, mrows)])

    return pl.kernel(
        body,
        out_type=jax.ShapeDtypeStruct((nc, _NP, _H), jnp.float32),
        mesh=mesh,
        scratch_types=(
            pltpu.VMEM((16, _CH), jnp.int32),
            pltpu.VMEM((16, _CH), jnp.int32),
            pltpu.VMEM((16, _CH), jnp.int32),
            pltpu.VMEM((_CH, _H), jnp.float32),
            pltpu.VMEM((_CH, _H), jnp.float32),
            pltpu.VMEM((_CH, _H), jnp.float32),
            pltpu.VMEM((_CH, _H), jnp.float32),
            pltpu.VMEM_SHARED((_NP, _H), jnp.float32),
            pltpu.SemaphoreType.DMA,
            pltpu.SemaphoreType.DMA,
            pltpu.SemaphoreType.DMA,
            pltpu.SemaphoreType.DMA,
            pltpu.SemaphoreType.DMA,
            pltpu.SemaphoreType.DMA,
        ),
    )


@functools.lru_cache(maxsize=None)
def _build_tc_layer(nc, with_res, final):
    blk = 256
    grid = _NP // blk

    def body(*refs):
        if with_res:
            h2, m, res, w, bv, g, bt, out_h, out_aux = refs
        else:
            h2, m, w, bv, g, bt, out_h, out_aux = refs
        t = h2[...]
        for c in range(nc):
            t = t + m[c]
        y = jnp.dot(t, w[...], preferred_element_type=jnp.float32) + bv[...]
        if with_res:
            y = y + res[...]
        out_h[...] = y
        mu = jnp.mean(y, axis=-1, keepdims=True)
        var = jnp.mean((y - mu) ** 2, axis=-1, keepdims=True)
        z = (y - mu) * lax.rsqrt(var + 1e-5) * g[...] + bt[...]
        if not final:
            z = jnp.maximum(z, 0.0)
        out_aux[...] = z

    row_spec = pl.BlockSpec((blk, _H), lambda i: (i, 0))
    m_spec = pl.BlockSpec((nc, blk, _H), lambda i: (0, i, 0))
    full_spec = pl.BlockSpec((_H, _H), lambda i: (0, 0))
    vec_spec = pl.BlockSpec((1, _H), lambda i: (0, 0))
    in_specs = [row_spec, m_spec]
    if with_res:
        in_specs.append(row_spec)
    in_specs += [full_spec, vec_spec, vec_spec, vec_spec]
    return pl.pallas_call(
        body,
        grid=(grid,),
        in_specs=in_specs,
        out_specs=[row_spec, row_spec],
        out_shape=[
            jax.ShapeDtypeStruct((_NP, _H), jnp.float32),
            jax.ShapeDtypeStruct((_NP, _H), jnp.float32),
        ],
    )


def kernel(x, edge_index, edge_attr, batch, atom_emb, bond_emb, W, b, ln_g, ln_b):
    nc, _ = _sc_info()
    x9r = jnp.pad(x, ((0, _NP - _N), (0, 0))).T.reshape(9 * (_NP // 40), 40)
    aembs = [atom_emb[f] for f in range(9)]
    bcat = bond_emb.reshape(24, _H)
    epad = _EP - _E
    src = jnp.pad(edge_index[0], (0, epad)).reshape(_ECHUNKS, _CH)
    dst = jnp.pad(edge_index[1], (0, epad),
                  constant_values=_N).reshape(_ECHUNKS, _CH)
    ea0 = jnp.pad(edge_attr[:, 0], (0, epad)).reshape(_ECHUNKS, _CH)
    ea1 = jnp.pad(edge_attr[:, 1], (0, epad)).reshape(_ECHUNKS, _CH)
    ea2 = jnp.pad(edge_attr[:, 2], (0, epad)).reshape(_ECHUNKS, _CH)

    sc_encode = _build_sc_encode()
    sc_msg = _build_sc_msg()

    henc, ctab, eidx = sc_encode(x9r, *aembs, bcat, ea0, ea1, ea2)

    m = sc_msg(henc, src, eidx, dst, ctab)
    h, aux = _build_tc_layer(nc, False, False)(
        henc, m, W[0], b[0:1], ln_g[0:1], ln_b[0:1])

    for l in range(1, _LAYERS):
        m = sc_msg(aux, src, eidx, dst, ctab)
        final = l == _LAYERS - 1
        h, aux = _build_tc_layer(nc, True, final)(
            aux, m, h, W[l], b[l:l + 1], ln_g[l:l + 1], ln_b[l:l + 1])

    return aux[:_N]


# speedup vs baseline: 1.1582x; 1.1582x over previous
import functools

import jax
import jax.numpy as jnp
from jax import lax
from jax.experimental import pallas as pl
from jax.experimental.pallas import tpu as pltpu
from jax.experimental.pallas import tpu_sc as plsc

_N = 10000
_NP = 10240
_E = 320000
_EP = 327680
_H = 128
_LAYERS = 7
_EPS = 1e-7
_CH = 64
_ECHUNKS = _EP // _CH


@functools.lru_cache(maxsize=None)
def _sc_info():
    info = plsc.get_sparse_core_info()
    return info.num_cores, info.num_subcores


@functools.lru_cache(maxsize=None)
def _build_sc_encode():
    nc, ns = _sc_info()
    ntiles = nc * ns
    nchunk = _ECHUNKS // ntiles
    rows_per_tile = _NP // ntiles
    xrows = _NP // 40
    crows = 512 // ntiles
    drows = _NP // ns
    mesh = plsc.VectorSubcoreMesh(core_axis_name="c", subcore_axis_name="s")

    def body(x9r, a0, a1, a2, a3, a4, a5, a6, a7, a8, bcat, ea0, ea1, ea2,
             henc, ctab, eidx,
             idxf, abuf, hacc, bb, ctb, e0v, e1v, e2v, sem):
        aembs = [a0, a1, a2, a3, a4, a5, a6, a7, a8]
        cid = lax.axis_index("c")
        sid = lax.axis_index("s")
        wid = cid * ns + sid

        pltpu.sync_copy(bcat, bb)

        @pl.loop(0, crows)
        def _(t):
            i = wid * crows + t
            r0 = i & 7
            r1 = 8 + ((i >> 3) & 7)
            r2 = 16 + ((i >> 6) & 7)
            for c in range(_H // 16):
                sl = pl.ds(c * 16, 16)
                ctb[t, sl] = bb[r0, sl] + bb[r1, sl] + bb[r2, sl]

        pltpu.sync_copy(ctb, ctab.at[pl.ds(wid * crows, crows)])

        ebase = wid * nchunk
        brows = nchunk // 4
        for blk in range(4):
            bb0 = ebase + blk * brows
            pltpu.sync_copy(ea0.at[pl.ds(bb0, brows)], e0v)
            pltpu.sync_copy(ea1.at[pl.ds(bb0, brows)], e1v)
            pltpu.sync_copy(ea2.at[pl.ds(bb0, brows)], e2v)

            @pl.loop(0, brows)
            def _(r):
                for c in range(_CH // 16):
                    sl = pl.ds(c * 16, 16)
                    e0v[r, sl] = (e0v[r, sl] + (e1v[r, sl] << 3)
                                  + (e2v[r, sl] << 6))

            pltpu.sync_copy(e0v, eidx.at[pl.ds(bb0, brows)])

        for f in range(9):
            pltpu.sync_copy(x9r.at[pl.ds(f * xrows + wid * 8, 8)], idxf)
            for k in range(8):
                if f == 0:
                    pltpu.async_copy(
                        aembs[f].at[idxf.at[k]],
                        hacc.at[pl.ds(k * 40, 40)], sem).wait()
                else:
                    pltpu.async_copy(aembs[f].at[idxf.at[k]], abuf, sem).wait()

                    @pl.loop(0, 40)
                    def _(r):
                        for c in range(_H // 16):
                            sl = pl.ds(c * 16, 16)
                            hacc[k * 40 + r, sl] = hacc[k * 40 + r, sl] + abuf[r, sl]

        pltpu.sync_copy(hacc, henc.at[pl.ds(wid * rows_per_tile, rows_per_tile)])

    return pl.kernel(
        body,
        out_type=(
            jax.ShapeDtypeStruct((_NP, _H), jnp.float32),
            jax.ShapeDtypeStruct((512, _H), jnp.float32),
            jax.ShapeDtypeStruct((_ECHUNKS, _CH), jnp.int32),
        ),
        mesh=mesh,
        scratch_types=(
            pltpu.VMEM((8, 40), jnp.int32),
            pltpu.VMEM((40, _H), jnp.float32),
            pltpu.VMEM((rows_per_tile, _H), jnp.float32),
            pltpu.VMEM((24, _H), jnp.float32),
            pltpu.VMEM((crows, _H), jnp.float32),
            pltpu.VMEM((nchunk // 4, _CH), jnp.int32),
            pltpu.VMEM((nchunk // 4, _CH), jnp.int32),
            pltpu.VMEM((nchunk // 4, _CH), jnp.int32),
            pltpu.SemaphoreType.DMA,
        ),
    )


@functools.lru_cache(maxsize=None)
def _build_sc_deg():
    nc, ns = _sc_info()
    ntiles = nc * ns
    nchunk = _ECHUNKS // ntiles
    drows = _NP // ns
    mesh = plsc.VectorSubcoreMesh(core_axis_name="c", subcore_axis_name="s")

    def body(dstr, dout, dvv, ones, dacc, dsem):
        cid = lax.axis_index("c")
        sid = lax.axis_index("s")
        wid = cid * ns + sid
        pltpu.sync_copy(dstr.at[pl.ds(wid * nchunk, nchunk)], dvv)

        @pl.loop(0, _CH)
        def _(r):
            for c in range(_H // 16):
                ones[r, pl.ds(c * 16, 16)] = jnp.zeros((16,), jnp.float32)

        dbase = sid * drows
        for k in range(drows // _CH):
            pltpu.sync_copy(ones, dacc.at[pl.ds(dbase + k * _CH, _CH)])

        @pl.loop(0, _CH)
        def _(r):
            for c in range(_H // 16):
                ones[r, pl.ds(c * 16, 16)] = jnp.full((16,), 1.0, jnp.float32)

        plsc.subcore_barrier()

        @pl.loop(0, 8)
        def _(r):
            pltpu.async_copy(ones, dacc.at[dvv.at[r]], dsem, add=True)

        @pl.loop(0, nchunk - 8)
        def _(r):
            pltpu.make_async_copy(ones, dacc.at[dvv.at[0]], dsem).wait()
            pltpu.async_copy(ones, dacc.at[dvv.at[r + 8]], dsem, add=True)

        @pl.loop(0, 8)
        def _(r):
            pltpu.make_async_copy(ones, dacc.at[dvv.at[0]], dsem).wait()

        plsc.subcore_barrier()
        pltpu.sync_copy(dacc.at[pl.ds(dbase, drows)],
                        dout.at[cid, pl.ds(dbase, drows)])

    return pl.kernel(
        body,
        out_type=jax.ShapeDtypeStruct((nc, _NP, _H), jnp.float32),
        mesh=mesh,
        scratch_types=(
            pltpu.VMEM((nchunk, _CH), jnp.int32),
            pltpu.VMEM((_CH, _H), jnp.float32),
            pltpu.VMEM_SHARED((_NP, _H), jnp.float32),
            pltpu.SemaphoreType.DMA,
        ),
    )


@functools.lru_cache(maxsize=None)
def _build_sc_msg():
    nc, ns = _sc_info()
    ntiles = nc * ns
    nchunk = _ECHUNKS // ntiles
    ngroup = nchunk // 8
    mrows = _NP // ns
    mesh = plsc.VectorSubcoreMesh(core_axis_name="c", subcore_axis_name="s")

    def body(hcur, srcr, eidxr, dstr, ctab,
             mout,
             sv, ev, dv, m0, m1, m2, m3, msh,
             g0, g1, g2, g3, s0, s1, s2, s3):
        cid = lax.axis_index("c")
        sid = lax.axis_index("s")
        wid = cid * ns + sid
        base = wid * nchunk
        mb = (m0, m1, m2, m3)
        gs = (g0, g1, g2, g3)
        ss = (s0, s1, s2, s3)

        def stage(grp, half):
            gb = base + grp * 8
            hs = pl.ds(half * 8, 8)
            pltpu.sync_copy(srcr.at[pl.ds(gb, 8)], sv.at[hs])
            pltpu.sync_copy(eidxr.at[pl.ds(gb, 8)], ev.at[hs])
            pltpu.sync_copy(dstr.at[pl.ds(gb, 8)], dv.at[hs])

        stage(0, 0)

        @pl.loop(0, _CH)
        def _(r):
            for c in range(_H // 16):
                m0[r, pl.ds(c * 16, 16)] = jnp.zeros((16,), jnp.float32)

        mbase = sid * mrows
        for k in range(mrows // _CH):
            pltpu.sync_copy(m0, msh.at[pl.ds(mbase + k * _CH, _CH)])
        plsc.subcore_barrier()

        pltpu.async_copy(hcur.at[sv.at[0]], m0, g0)
        pltpu.make_async_copy(hcur.at[sv.at[0]], m0, g0).wait()
        pltpu.async_copy(ctab.at[ev.at[0]], m0, g0, add=True)
        pltpu.async_copy(hcur.at[sv.at[1]], m1, g1)

        @pl.loop(0, ngroup)
        def _(g):
            par = g & 1
            par2 = (g + 1) & 1
            for j in range(8):
                b = j & 3
                b1 = (j + 1) & 3
                b2 = (j + 2) & 3
                c = g * 8 + j
                if j < 7:
                    row1 = par * 8 + j + 1
                else:
                    row1 = par2 * 8
                if j < 6:
                    row2 = par * 8 + j + 2
                else:
                    row2 = par2 * 8 + (j - 6)

                @pl.when(c < nchunk - 1)
                def _():
                    pltpu.make_async_copy(
                        hcur.at[sv.at[row1]], mb[b1], gs[b1]).wait()
                    pltpu.async_copy(ctab.at[ev.at[row1]], mb[b1], gs[b1],
                                     add=True)

                @pl.when(c < nchunk - 2)
                def _():
                    @pl.when(c >= 2)
                    def _():
                        pltpu.make_async_copy(
                            mb[b2], msh.at[dv.at[0]], ss[b2]).wait()
                    pltpu.async_copy(hcur.at[sv.at[row2]], mb[b2], gs[b2])

                if j == 2:
                    @pl.when(g < ngroup - 1)
                    def _():
                        stage(g + 1, par2)

                pltpu.make_async_copy(ctab.at[ev.at[0]], mb[b], gs[b]).wait()

                @pl.loop(0, _CH)
                def _(r):
                    for cc in range(_H // 16):
                        sl = pl.ds(cc * 16, 16)
                        mb[b][r, sl] = jnp.maximum(mb[b][r, sl], 0.0)

                pltpu.async_copy(mb[b], msh.at[dv.at[par * 8 + j]], ss[b],
                                 add=True)

        for k in range(4):
            pltpu.make_async_copy(mb[k], msh.at[dv.at[0]], ss[k]).wait()
        plsc.subcore_barrier()
        pltpu.sync_copy(msh.at[pl.ds(mbase, mrows)],
                        mout.at[cid, pl.ds(mbase, mrows)])

    return pl.kernel(
        body,
        out_type=jax.ShapeDtypeStruct((nc, _NP, _H), jnp.float32),
        mesh=mesh,
        scratch_types=(
            pltpu.VMEM((16, _CH), jnp.int32),
            pltpu.VMEM((16, _CH), jnp.int32),
            pltpu.VMEM((16, _CH), jnp.int32),
            pltpu.VMEM((_CH, _H), jnp.float32),
            pltpu.VMEM((_CH, _H), jnp.float32),
            pltpu.VMEM((_CH, _H), jnp.float32),
            pltpu.VMEM((_CH, _H), jnp.float32),
            pltpu.VMEM_SHARED((_NP, _H), jnp.float32),
            pltpu.SemaphoreType.DMA,
            pltpu.SemaphoreType.DMA,
            pltpu.SemaphoreType.DMA,
            pltpu.SemaphoreType.DMA,
            pltpu.SemaphoreType.DMA,
            pltpu.SemaphoreType.DMA,
            pltpu.SemaphoreType.DMA,
            pltpu.SemaphoreType.DMA,
        ),
    )


@functools.lru_cache(maxsize=None)
def _build_tc_layer(nc, with_res, final):
    blk = 256
    grid = _NP // blk

    def body(*refs):
        if with_res:
            h2, m, degf, res, w, bv, g, bt, out_h, out_aux = refs
        else:
            h2, m, degf, w, bv, g, bt, out_h, out_aux = refs
        t = h2[...] + degf[...]
        for c in range(nc):
            t = t + m[c]
        y = jnp.dot(t, w[...], preferred_element_type=jnp.float32) + bv[...]
        if with_res:
            y = y + res[...]
        out_h[...] = y
        mu = jnp.mean(y, axis=-1, keepdims=True)
        var = jnp.mean((y - mu) ** 2, axis=-1, keepdims=True)
        z = (y - mu) * lax.rsqrt(var + 1e-5) * g[...] + bt[...]
        if not final:
            z = jnp.maximum(z, 0.0)
        out_aux[...] = z

    row_spec = pl.BlockSpec((blk, _H), lambda i: (i, 0))
    m_spec = pl.BlockSpec((nc, blk, _H), lambda i: (0, i, 0))
    full_spec = pl.BlockSpec((_H, _H), lambda i: (0, 0))
    vec_spec = pl.BlockSpec((1, _H), lambda i: (0, 0))
    in_specs = [row_spec, m_spec, row_spec]
    if with_res:
        in_specs.append(row_spec)
    in_specs += [full_spec, vec_spec, vec_spec, vec_spec]
    return pl.pallas_call(
        body,
        grid=(grid,),
        in_specs=in_specs,
        out_specs=[row_spec, row_spec],
        out_shape=[
            jax.ShapeDtypeStruct((_NP, _H), jnp.float32),
            jax.ShapeDtypeStruct((_NP, _H), jnp.float32),
        ],
    )


def kernel(x, edge_index, edge_attr, batch, atom_emb, bond_emb, W, b, ln_g, ln_b):
    nc, _ = _sc_info()
    x9r = jnp.pad(x, ((0, _NP - _N), (0, 0))).T.reshape(9 * (_NP // 40), 40)
    aembs = [atom_emb[f] for f in range(9)]
    bcat = bond_emb.reshape(24, _H)
    epad = _EP - _E
    src = jnp.pad(edge_index[0], (0, epad)).reshape(_ECHUNKS, _CH)
    dst = jnp.pad(edge_index[1], (0, epad),
                  constant_values=_N).reshape(_ECHUNKS, _CH)
    ea0 = jnp.pad(edge_attr[:, 0], (0, epad)).reshape(_ECHUNKS, _CH)
    ea1 = jnp.pad(edge_attr[:, 1], (0, epad)).reshape(_ECHUNKS, _CH)
    ea2 = jnp.pad(edge_attr[:, 2], (0, epad)).reshape(_ECHUNKS, _CH)

    sc_encode = _build_sc_encode()
    sc_deg = _build_sc_deg()
    sc_msg = _build_sc_msg()

    henc, ctab, eidx = sc_encode(x9r, *aembs, bcat, ea0, ea1, ea2)
    deg = sc_deg(dst)
    degf = _EPS * (deg[0] + deg[1]) if nc == 2 else _EPS * deg.sum(axis=0)

    m = sc_msg(henc, src, eidx, dst, ctab)
    h, aux = _build_tc_layer(nc, False, False)(
        henc, m, degf, W[0], b[0:1], ln_g[0:1], ln_b[0:1])

    for l in range(1, _LAYERS):
        m = sc_msg(aux, src, eidx, dst, ctab)
        final = l == _LAYERS - 1
        h, aux = _build_tc_layer(nc, True, final)(
            aux, m, degf, h, W[l], b[l:l + 1], ln_g[l:l + 1], ln_b[l:l + 1])

    return aux[:_N]


# speedup vs baseline: 1.5264x; 1.3179x over previous
import functools

import jax
import jax.numpy as jnp
from jax import lax
from jax.experimental import pallas as pl
from jax.experimental.pallas import tpu as pltpu
from jax.experimental.pallas import tpu_sc as plsc

_N = 10000
_NP = 10240
_E = 320000
_EP = 327680
_H = 128
_LAYERS = 7
_EPS = 1e-7
_CH = 64
_ECHUNKS = _EP // _CH


@functools.lru_cache(maxsize=None)
def _sc_info():
    info = plsc.get_sparse_core_info()
    return info.num_cores, info.num_subcores


@functools.lru_cache(maxsize=None)
def _build_sc_encode():
    nc, ns = _sc_info()
    ntiles = nc * ns
    nchunk = _ECHUNKS // ntiles
    rows_per_tile = _NP // ntiles
    xrows = _NP // 40
    crows = 512 // ntiles
    drows = _NP // ns
    mesh = plsc.VectorSubcoreMesh(core_axis_name="c", subcore_axis_name="s")

    def body(x9r, a0, a1, a2, a3, a4, a5, a6, a7, a8, bcat, ea0, ea1, ea2,
             henc, ctab, eidx,
             idxf, abuf, hacc, bb, ctb, e0v, e1v, e2v, sem):
        aembs = [a0, a1, a2, a3, a4, a5, a6, a7, a8]
        cid = lax.axis_index("c")
        sid = lax.axis_index("s")
        wid = cid * ns + sid

        pltpu.sync_copy(bcat, bb)

        @pl.loop(0, crows)
        def _(t):
            i = wid * crows + t
            r0 = i & 7
            r1 = 8 + ((i >> 3) & 7)
            r2 = 16 + ((i >> 6) & 7)
            for c in range(_H // 16):
                sl = pl.ds(c * 16, 16)
                ctb[t, sl] = bb[r0, sl] + bb[r1, sl] + bb[r2, sl]

        pltpu.sync_copy(ctb, ctab.at[pl.ds(wid * crows, crows)])

        ebase = wid * nchunk
        brows = nchunk // 4
        for blk in range(4):
            bb0 = ebase + blk * brows
            pltpu.sync_copy(ea0.at[pl.ds(bb0, brows)], e0v)
            pltpu.sync_copy(ea1.at[pl.ds(bb0, brows)], e1v)
            pltpu.sync_copy(ea2.at[pl.ds(bb0, brows)], e2v)

            @pl.loop(0, brows)
            def _(r):
                for c in range(_CH // 16):
                    sl = pl.ds(c * 16, 16)
                    e0v[r, sl] = (e0v[r, sl] + (e1v[r, sl] << 3)
                                  + (e2v[r, sl] << 6))

            pltpu.sync_copy(e0v, eidx.at[pl.ds(bb0, brows)])

        for f in range(9):
            pltpu.sync_copy(x9r.at[pl.ds(f * xrows + wid * 8, 8)], idxf)
            for k in range(8):
                if f == 0:
                    pltpu.async_copy(
                        aembs[f].at[idxf.at[k]],
                        hacc.at[pl.ds(k * 40, 40)], sem).wait()
                else:
                    pltpu.async_copy(aembs[f].at[idxf.at[k]], abuf, sem).wait()

                    @pl.loop(0, 40)
                    def _(r):
                        for c in range(_H // 16):
                            sl = pl.ds(c * 16, 16)
                            hacc[k * 40 + r, sl] = hacc[k * 40 + r, sl] + abuf[r, sl]

        pltpu.sync_copy(hacc, henc.at[pl.ds(wid * rows_per_tile, rows_per_tile)])

    return pl.kernel(
        body,
        out_type=(
            jax.ShapeDtypeStruct((_NP, _H), jnp.float32),
            jax.ShapeDtypeStruct((512, _H), jnp.float32),
            jax.ShapeDtypeStruct((_ECHUNKS, _CH), jnp.int32),
        ),
        mesh=mesh,
        scratch_types=(
            pltpu.VMEM((8, 40), jnp.int32),
            pltpu.VMEM((40, _H), jnp.float32),
            pltpu.VMEM((rows_per_tile, _H), jnp.float32),
            pltpu.VMEM((24, _H), jnp.float32),
            pltpu.VMEM((crows, _H), jnp.float32),
            pltpu.VMEM((nchunk // 4, _CH), jnp.int32),
            pltpu.VMEM((nchunk // 4, _CH), jnp.int32),
            pltpu.VMEM((nchunk // 4, _CH), jnp.int32),
            pltpu.SemaphoreType.DMA,
        ),
    )


@functools.lru_cache(maxsize=None)
def _build_sc_deg():
    nc, ns = _sc_info()
    ntiles = nc * ns
    nchunk = _ECHUNKS // ntiles
    drows = _NP // ns
    mesh = plsc.VectorSubcoreMesh(core_axis_name="c", subcore_axis_name="s")

    def body(dstr, dout, dvv, ones, dacc, dsem):
        cid = lax.axis_index("c")
        sid = lax.axis_index("s")
        wid = cid * ns + sid
        pltpu.sync_copy(dstr.at[pl.ds(wid * nchunk, nchunk)], dvv)

        @pl.loop(0, _CH)
        def _(r):
            for c in range(_H // 16):
                ones[r, pl.ds(c * 16, 16)] = jnp.zeros((16,), jnp.float32)

        dbase = sid * drows
        for k in range(drows // _CH):
            pltpu.sync_copy(ones, dacc.at[pl.ds(dbase + k * _CH, _CH)])

        @pl.loop(0, _CH)
        def _(r):
            for c in range(_H // 16):
                ones[r, pl.ds(c * 16, 16)] = jnp.full((16,), 1.0, jnp.float32)

        plsc.subcore_barrier()

        @pl.loop(0, 8)
        def _(r):
            pltpu.async_copy(ones, dacc.at[dvv.at[r]], dsem, add=True)

        @pl.loop(0, nchunk - 8)
        def _(r):
            pltpu.make_async_copy(ones, dacc.at[dvv.at[0]], dsem).wait()
            pltpu.async_copy(ones, dacc.at[dvv.at[r + 8]], dsem, add=True)

        @pl.loop(0, 8)
        def _(r):
            pltpu.make_async_copy(ones, dacc.at[dvv.at[0]], dsem).wait()

        plsc.subcore_barrier()
        pltpu.sync_copy(dacc.at[pl.ds(dbase, drows)],
                        dout.at[cid, pl.ds(dbase, drows)])

    return pl.kernel(
        body,
        out_type=jax.ShapeDtypeStruct((nc, _NP, _H), jnp.float32),
        mesh=mesh,
        scratch_types=(
            pltpu.VMEM((nchunk, _CH), jnp.int32),
            pltpu.VMEM((_CH, _H), jnp.float32),
            pltpu.VMEM_SHARED((_NP, _H), jnp.float32),
            pltpu.SemaphoreType.DMA,
        ),
    )


@functools.lru_cache(maxsize=None)
def _build_sc_msg():
    nc, ns = _sc_info()
    ntiles = nc * ns
    nchunk = _ECHUNKS // ntiles
    ngroup = nchunk // 8
    mrows = _NP // ns
    crows = 512 // ns
    mesh = plsc.VectorSubcoreMesh(core_axis_name="c", subcore_axis_name="s")

    def body(hcur, srcr, eidxr, dstr, ctab,
             mout,
             sv, ev, dv, m0, m1, m2, m3, msh, csh,
             g0, g1, g2, g3, s0, s1, s2, s3):
        cid = lax.axis_index("c")
        sid = lax.axis_index("s")
        wid = cid * ns + sid
        base = wid * nchunk
        mb = (m0, m1, m2, m3)
        gs = (g0, g1, g2, g3)
        ss = (s0, s1, s2, s3)

        def stage(grp, half):
            gb = base + grp * 8
            hs = pl.ds(half * 8, 8)
            pltpu.sync_copy(srcr.at[pl.ds(gb, 8)], sv.at[hs])
            pltpu.sync_copy(eidxr.at[pl.ds(gb, 8)], ev.at[hs])
            pltpu.sync_copy(dstr.at[pl.ds(gb, 8)], dv.at[hs])

        stage(0, 0)

        @pl.loop(0, _CH)
        def _(r):
            for c in range(_H // 16):
                m0[r, pl.ds(c * 16, 16)] = jnp.zeros((16,), jnp.float32)

        pltpu.sync_copy(ctab.at[pl.ds(sid * crows, crows)],
                        csh.at[pl.ds(sid * crows, crows)])
        mbase = sid * mrows
        for k in range(mrows // _CH):
            pltpu.sync_copy(m0, msh.at[pl.ds(mbase + k * _CH, _CH)])
        plsc.subcore_barrier()

        pltpu.async_copy(hcur.at[sv.at[0]], m0, g0)
        pltpu.make_async_copy(hcur.at[sv.at[0]], m0, g0).wait()
        pltpu.async_copy(csh.at[ev.at[0]], m0, g0, add=True)
        pltpu.async_copy(hcur.at[sv.at[1]], m1, g1)

        @pl.loop(0, ngroup)
        def _(g):
            par = g & 1
            par2 = (g + 1) & 1
            for j in range(8):
                b = j & 3
                b1 = (j + 1) & 3
                b2 = (j + 2) & 3
                c = g * 8 + j
                if j < 7:
                    row1 = par * 8 + j + 1
                else:
                    row1 = par2 * 8
                if j < 6:
                    row2 = par * 8 + j + 2
                else:
                    row2 = par2 * 8 + (j - 6)

                @pl.when(c < nchunk - 1)
                def _():
                    pltpu.make_async_copy(
                        hcur.at[sv.at[row1]], mb[b1], gs[b1]).wait()
                    pltpu.async_copy(csh.at[ev.at[row1]], mb[b1], gs[b1],
                                     add=True)

                @pl.when(c < nchunk - 2)
                def _():
                    @pl.when(c >= 2)
                    def _():
                        pltpu.make_async_copy(
                            mb[b2], msh.at[dv.at[0]], ss[b2]).wait()
                    pltpu.async_copy(hcur.at[sv.at[row2]], mb[b2], gs[b2])

                if j == 2:
                    @pl.when(g < ngroup - 1)
                    def _():
                        stage(g + 1, par2)

                pltpu.make_async_copy(csh.at[ev.at[0]], mb[b], gs[b]).wait()

                @pl.loop(0, _CH)
                def _(r):
                    for cc in range(_H // 16):
                        sl = pl.ds(cc * 16, 16)
                        mb[b][r, sl] = jnp.maximum(mb[b][r, sl], 0.0)

                pltpu.async_copy(mb[b], msh.at[dv.at[par * 8 + j]], ss[b],
                                 add=True)

        for k in range(4):
            pltpu.make_async_copy(mb[k], msh.at[dv.at[0]], ss[k]).wait()
        plsc.subcore_barrier()
        pltpu.sync_copy(msh.at[pl.ds(mbase, mrows)],
                        mout.at[cid, pl.ds(mbase, mrows)])

    return pl.kernel(
        body,
        out_type=jax.ShapeDtypeStruct((nc, _NP, _H), jnp.float32),
        mesh=mesh,
        scratch_types=(
            pltpu.VMEM((16, _CH), jnp.int32),
            pltpu.VMEM((16, _CH), jnp.int32),
            pltpu.VMEM((16, _CH), jnp.int32),
            pltpu.VMEM((_CH, _H), jnp.float32),
            pltpu.VMEM((_CH, _H), jnp.float32),
            pltpu.VMEM((_CH, _H), jnp.float32),
            pltpu.VMEM((_CH, _H), jnp.float32),
            pltpu.VMEM_SHARED((_NP, _H), jnp.float32),
            pltpu.VMEM_SHARED((512, _H), jnp.float32),
            pltpu.SemaphoreType.DMA,
            pltpu.SemaphoreType.DMA,
            pltpu.SemaphoreType.DMA,
            pltpu.SemaphoreType.DMA,
            pltpu.SemaphoreType.DMA,
            pltpu.SemaphoreType.DMA,
            pltpu.SemaphoreType.DMA,
            pltpu.SemaphoreType.DMA,
        ),
    )


@functools.lru_cache(maxsize=None)
def _build_tc_layer(nc, with_res, final):
    blk = 256
    grid = _NP // blk

    def body(*refs):
        if with_res:
            h2, m, degf, res, w, bv, g, bt, out_h, out_aux = refs
        else:
            h2, m, degf, w, bv, g, bt, out_h, out_aux = refs
        t = h2[...] + degf[...]
        for c in range(nc):
            t = t + m[c]
        y = jnp.dot(t, w[...], preferred_element_type=jnp.float32) + bv[...]
        if with_res:
            y = y + res[...]
        out_h[...] = y
        mu = jnp.mean(y, axis=-1, keepdims=True)
        var = jnp.mean((y - mu) ** 2, axis=-1, keepdims=True)
        z = (y - mu) * lax.rsqrt(var + 1e-5) * g[...] + bt[...]
        if not final:
            z = jnp.maximum(z, 0.0)
        out_aux[...] = z

    row_spec = pl.BlockSpec((blk, _H), lambda i: (i, 0))
    m_spec = pl.BlockSpec((nc, blk, _H), lambda i: (0, i, 0))
    full_spec = pl.BlockSpec((_H, _H), lambda i: (0, 0))
    vec_spec = pl.BlockSpec((1, _H), lambda i: (0, 0))
    in_specs = [row_spec, m_spec, row_spec]
    if with_res:
        in_specs.append(row_spec)
    in_specs += [full_spec, vec_spec, vec_spec, vec_spec]
    return pl.pallas_call(
        body,
        grid=(grid,),
        in_specs=in_specs,
        out_specs=[row_spec, row_spec],
        out_shape=[
            jax.ShapeDtypeStruct((_NP, _H), jnp.float32),
            jax.ShapeDtypeStruct((_NP, _H), jnp.float32),
        ],
    )


def kernel(x, edge_index, edge_attr, batch, atom_emb, bond_emb, W, b, ln_g, ln_b):
    nc, _ = _sc_info()
    x9r = jnp.pad(x, ((0, _NP - _N), (0, 0))).T.reshape(9 * (_NP // 40), 40)
    aembs = [atom_emb[f] for f in range(9)]
    bcat = bond_emb.reshape(24, _H)
    epad = _EP - _E
    src = jnp.pad(edge_index[0], (0, epad)).reshape(_ECHUNKS, _CH)
    dst = jnp.pad(edge_index[1], (0, epad),
                  constant_values=_N).reshape(_ECHUNKS, _CH)
    ea0 = jnp.pad(edge_attr[:, 0], (0, epad)).reshape(_ECHUNKS, _CH)
    ea1 = jnp.pad(edge_attr[:, 1], (0, epad)).reshape(_ECHUNKS, _CH)
    ea2 = jnp.pad(edge_attr[:, 2], (0, epad)).reshape(_ECHUNKS, _CH)

    sc_encode = _build_sc_encode()
    sc_deg = _build_sc_deg()
    sc_msg = _build_sc_msg()

    henc, ctab, eidx = sc_encode(x9r, *aembs, bcat, ea0, ea1, ea2)
    deg = sc_deg(dst)
    degf = _EPS * (deg[0] + deg[1]) if nc == 2 else _EPS * deg.sum(axis=0)

    m = sc_msg(henc, src, eidx, dst, ctab)
    h, aux = _build_tc_layer(nc, False, False)(
        henc, m, degf, W[0], b[0:1], ln_g[0:1], ln_b[0:1])

    for l in range(1, _LAYERS):
        m = sc_msg(aux, src, eidx, dst, ctab)
        final = l == _LAYERS - 1
        h, aux = _build_tc_layer(nc, True, final)(
            aux, m, degf, h, W[l], b[l:l + 1], ln_g[l:l + 1], ln_b[l:l + 1])

    return aux[:_N]


# speedup vs baseline: 1.5404x; 1.0092x over previous
import functools

import jax
import jax.numpy as jnp
from jax import lax
from jax.experimental import pallas as pl
from jax.experimental.pallas import tpu as pltpu
from jax.experimental.pallas import tpu_sc as plsc

_N = 10000
_NP = 10240
_E = 320000
_EP = 327680
_H = 128
_LAYERS = 7
_EPS = 1e-7
_CH = 64
_ECHUNKS = _EP // _CH


@functools.lru_cache(maxsize=None)
def _sc_info():
    info = plsc.get_sparse_core_info()
    return info.num_cores, info.num_subcores


@functools.lru_cache(maxsize=None)
def _build_sc_encode():
    nc, ns = _sc_info()
    ntiles = nc * ns
    nchunk = _ECHUNKS // ntiles
    rows_per_tile = _NP // ntiles
    xrows = _NP // 40
    crows = 512 // ntiles
    drows = _NP // ns
    mesh = plsc.VectorSubcoreMesh(core_axis_name="c", subcore_axis_name="s")

    def body(x9r, a0, a1, a2, a3, a4, a5, a6, a7, a8, bcat, ea0, ea1, ea2,
             henc, ctab, eidx,
             idxf, abuf, hacc, bb, ctb, e0v, e1v, e2v, sem):
        aembs = [a0, a1, a2, a3, a4, a5, a6, a7, a8]
        cid = lax.axis_index("c")
        sid = lax.axis_index("s")
        wid = cid * ns + sid

        pltpu.sync_copy(bcat, bb)

        @pl.loop(0, crows)
        def _(t):
            i = wid * crows + t
            r0 = i & 7
            r1 = 8 + ((i >> 3) & 7)
            r2 = 16 + ((i >> 6) & 7)
            for c in range(_H // 16):
                sl = pl.ds(c * 16, 16)
                ctb[t, sl] = bb[r0, sl] + bb[r1, sl] + bb[r2, sl]

        pltpu.sync_copy(ctb, ctab.at[pl.ds(wid * crows, crows)])

        ebase = wid * nchunk
        brows = nchunk // 4
        for blk in range(4):
            bb0 = ebase + blk * brows
            pltpu.sync_copy(ea0.at[pl.ds(bb0, brows)], e0v)
            pltpu.sync_copy(ea1.at[pl.ds(bb0, brows)], e1v)
            pltpu.sync_copy(ea2.at[pl.ds(bb0, brows)], e2v)

            @pl.loop(0, brows)
            def _(r):
                for c in range(_CH // 16):
                    sl = pl.ds(c * 16, 16)
                    e0v[r, sl] = (e0v[r, sl] + (e1v[r, sl] << 3)
                                  + (e2v[r, sl] << 6))

            pltpu.sync_copy(e0v, eidx.at[pl.ds(bb0, brows)])

        for f in range(9):
            pltpu.sync_copy(x9r.at[pl.ds(f * xrows + wid * 8, 8)], idxf)
            for k in range(8):
                if f == 0:
                    pltpu.async_copy(
                        aembs[f].at[idxf.at[k]],
                        hacc.at[pl.ds(k * 40, 40)], sem).wait()
                else:
                    pltpu.async_copy(aembs[f].at[idxf.at[k]], abuf, sem).wait()

                    @pl.loop(0, 40)
                    def _(r):
                        for c in range(_H // 16):
                            sl = pl.ds(c * 16, 16)
                            hacc[k * 40 + r, sl] = hacc[k * 40 + r, sl] + abuf[r, sl]

        pltpu.sync_copy(hacc, henc.at[pl.ds(wid * rows_per_tile, rows_per_tile)])

    return pl.kernel(
        body,
        out_type=(
            jax.ShapeDtypeStruct((_NP, _H), jnp.float32),
            jax.ShapeDtypeStruct((512, _H), jnp.float32),
            jax.ShapeDtypeStruct((_ECHUNKS, _CH), jnp.int32),
        ),
        mesh=mesh,
        scratch_types=(
            pltpu.VMEM((8, 40), jnp.int32),
            pltpu.VMEM((40, _H), jnp.float32),
            pltpu.VMEM((rows_per_tile, _H), jnp.float32),
            pltpu.VMEM((24, _H), jnp.float32),
            pltpu.VMEM((crows, _H), jnp.float32),
            pltpu.VMEM((nchunk // 4, _CH), jnp.int32),
            pltpu.VMEM((nchunk // 4, _CH), jnp.int32),
            pltpu.VMEM((nchunk // 4, _CH), jnp.int32),
            pltpu.SemaphoreType.DMA,
        ),
    )


@functools.lru_cache(maxsize=None)
def _build_sc_deg():
    nc, ns = _sc_info()
    ntiles = nc * ns
    nchunk = _ECHUNKS // ntiles
    drows = _NP // ns
    mesh = plsc.VectorSubcoreMesh(core_axis_name="c", subcore_axis_name="s")

    def body(dstr, dout, dvv, ones, dacc, dsem):
        cid = lax.axis_index("c")
        sid = lax.axis_index("s")
        wid = cid * ns + sid
        pltpu.sync_copy(dstr.at[pl.ds(wid * nchunk, nchunk)], dvv)

        @pl.loop(0, _CH)
        def _(r):
            ones[r, pl.ds(0, 16)] = jnp.zeros((16,), jnp.float32)

        dbase = sid * drows
        for k in range(drows // _CH):
            pltpu.sync_copy(ones, dacc.at[pl.ds(dbase + k * _CH, _CH)])

        @pl.loop(0, _CH)
        def _(r):
            ones[r, pl.ds(0, 16)] = jnp.full((16,), 1.0, jnp.float32)

        plsc.subcore_barrier()

        @pl.loop(0, 8)
        def _(r):
            pltpu.async_copy(ones, dacc.at[dvv.at[r]], dsem, add=True)

        @pl.loop(0, nchunk - 8)
        def _(r):
            pltpu.make_async_copy(ones, dacc.at[dvv.at[0]], dsem).wait()
            pltpu.async_copy(ones, dacc.at[dvv.at[r + 8]], dsem, add=True)

        @pl.loop(0, 8)
        def _(r):
            pltpu.make_async_copy(ones, dacc.at[dvv.at[0]], dsem).wait()

        plsc.subcore_barrier()
        pltpu.sync_copy(dacc.at[pl.ds(dbase, drows)],
                        dout.at[cid, pl.ds(dbase, drows)])

    return pl.kernel(
        body,
        out_type=jax.ShapeDtypeStruct((nc, _NP, 16), jnp.float32),
        mesh=mesh,
        scratch_types=(
            pltpu.VMEM((nchunk, _CH), jnp.int32),
            pltpu.VMEM((_CH, 16), jnp.float32),
            pltpu.VMEM_SHARED((_NP, 16), jnp.float32),
            pltpu.SemaphoreType.DMA,
        ),
    )


@functools.lru_cache(maxsize=None)
def _build_sc_msg():
    nc, ns = _sc_info()
    ntiles = nc * ns
    nchunk = _ECHUNKS // ntiles
    ngroup = nchunk // 8
    mrows = _NP // ns
    crows = 512 // ns
    mesh = plsc.VectorSubcoreMesh(core_axis_name="c", subcore_axis_name="s")

    def body(hcur, srcr, eidxr, dstr, ctab,
             mout,
             sv, ev, dv, m0, m1, m2, m3, msh, csh,
             g0, g1, g2, g3, s0, s1, s2, s3):
        cid = lax.axis_index("c")
        sid = lax.axis_index("s")
        wid = cid * ns + sid
        base = wid * nchunk
        mb = (m0, m1, m2, m3)
        gs = (g0, g1, g2, g3)
        ss = (s0, s1, s2, s3)

        def stage(grp, half):
            gb = base + grp * 8
            hs = pl.ds(half * 8, 8)
            pltpu.sync_copy(srcr.at[pl.ds(gb, 8)], sv.at[hs])
            pltpu.sync_copy(eidxr.at[pl.ds(gb, 8)], ev.at[hs])
            pltpu.sync_copy(dstr.at[pl.ds(gb, 8)], dv.at[hs])

        stage(0, 0)

        @pl.loop(0, _CH)
        def _(r):
            for c in range(_H // 16):
                m0[r, pl.ds(c * 16, 16)] = jnp.zeros((16,), jnp.float32)

        pltpu.sync_copy(ctab.at[pl.ds(sid * crows, crows)],
                        csh.at[pl.ds(sid * crows, crows)])
        mbase = sid * mrows
        for k in range(mrows // _CH):
            pltpu.sync_copy(m0, msh.at[pl.ds(mbase + k * _CH, _CH)])
        plsc.subcore_barrier()

        pltpu.async_copy(hcur.at[sv.at[0]], m0, g0)
        pltpu.make_async_copy(hcur.at[sv.at[0]], m0, g0).wait()
        pltpu.async_copy(csh.at[ev.at[0]], m0, g0, add=True)
        pltpu.async_copy(hcur.at[sv.at[1]], m1, g1)

        @pl.loop(0, ngroup)
        def _(g):
            par = g & 1
            par2 = (g + 1) & 1
            for j in range(8):
                b = j & 3
                b1 = (j + 1) & 3
                b2 = (j + 2) & 3
                c = g * 8 + j
                if j < 7:
                    row1 = par * 8 + j + 1
                else:
                    row1 = par2 * 8
                if j < 6:
                    row2 = par * 8 + j + 2
                else:
                    row2 = par2 * 8 + (j - 6)

                @pl.when(c < nchunk - 1)
                def _():
                    pltpu.make_async_copy(
                        hcur.at[sv.at[row1]], mb[b1], gs[b1]).wait()
                    pltpu.async_copy(csh.at[ev.at[row1]], mb[b1], gs[b1],
                                     add=True)

                @pl.when(c < nchunk - 2)
                def _():
                    @pl.when(c >= 2)
                    def _():
                        pltpu.make_async_copy(
                            mb[b2], msh.at[dv.at[0]], ss[b2]).wait()
                    pltpu.async_copy(hcur.at[sv.at[row2]], mb[b2], gs[b2])

                if j == 2:
                    @pl.when(g < ngroup - 1)
                    def _():
                        stage(g + 1, par2)

                pltpu.make_async_copy(csh.at[ev.at[0]], mb[b], gs[b]).wait()

                @pl.loop(0, _CH)
                def _(r):
                    for cc in range(_H // 16):
                        sl = pl.ds(cc * 16, 16)
                        mb[b][r, sl] = jnp.maximum(mb[b][r, sl], 0.0)

                pltpu.async_copy(mb[b], msh.at[dv.at[par * 8 + j]], ss[b],
                                 add=True)

        for k in range(4):
            pltpu.make_async_copy(mb[k], msh.at[dv.at[0]], ss[k]).wait()
        plsc.subcore_barrier()
        pltpu.sync_copy(msh.at[pl.ds(mbase, mrows)],
                        mout.at[cid, pl.ds(mbase, mrows)])

    return pl.kernel(
        body,
        out_type=jax.ShapeDtypeStruct((nc, _NP, _H), jnp.float32),
        mesh=mesh,
        scratch_types=(
            pltpu.VMEM((16, _CH), jnp.int32),
            pltpu.VMEM((16, _CH), jnp.int32),
            pltpu.VMEM((16, _CH), jnp.int32),
            pltpu.VMEM((_CH, _H), jnp.float32),
            pltpu.VMEM((_CH, _H), jnp.float32),
            pltpu.VMEM((_CH, _H), jnp.float32),
            pltpu.VMEM((_CH, _H), jnp.float32),
            pltpu.VMEM_SHARED((_NP, _H), jnp.float32),
            pltpu.VMEM_SHARED((512, _H), jnp.float32),
            pltpu.SemaphoreType.DMA,
            pltpu.SemaphoreType.DMA,
            pltpu.SemaphoreType.DMA,
            pltpu.SemaphoreType.DMA,
            pltpu.SemaphoreType.DMA,
            pltpu.SemaphoreType.DMA,
            pltpu.SemaphoreType.DMA,
            pltpu.SemaphoreType.DMA,
        ),
    )


@functools.lru_cache(maxsize=None)
def _build_tc_layer(nc, with_res, final):
    blk = 256
    grid = _NP // blk

    def body(*refs):
        if with_res:
            h2, m, degf, res, w, bv, g, bt, out_h, out_aux = refs
        else:
            h2, m, degf, w, bv, g, bt, out_h, out_aux = refs
        t = h2[...] + degf[..., 0:1]
        for c in range(nc):
            t = t + m[c]
        y = jnp.dot(t, w[...], preferred_element_type=jnp.float32) + bv[...]
        if with_res:
            y = y + res[...]
        out_h[...] = y
        mu = jnp.mean(y, axis=-1, keepdims=True)
        var = jnp.mean((y - mu) ** 2, axis=-1, keepdims=True)
        z = (y - mu) * lax.rsqrt(var + 1e-5) * g[...] + bt[...]
        if not final:
            z = jnp.maximum(z, 0.0)
        out_aux[...] = z

    row_spec = pl.BlockSpec((blk, _H), lambda i: (i, 0))
    m_spec = pl.BlockSpec((nc, blk, _H), lambda i: (0, i, 0))
    full_spec = pl.BlockSpec((_H, _H), lambda i: (0, 0))
    vec_spec = pl.BlockSpec((1, _H), lambda i: (0, 0))
    deg_spec = pl.BlockSpec((blk, 16), lambda i: (i, 0))
    in_specs = [row_spec, m_spec, deg_spec]
    if with_res:
        in_specs.append(row_spec)
    in_specs += [full_spec, vec_spec, vec_spec, vec_spec]
    return pl.pallas_call(
        body,
        grid=(grid,),
        in_specs=in_specs,
        out_specs=[row_spec, row_spec],
        out_shape=[
            jax.ShapeDtypeStruct((_NP, _H), jnp.float32),
            jax.ShapeDtypeStruct((_NP, _H), jnp.float32),
        ],
    )


def kernel(x, edge_index, edge_attr, batch, atom_emb, bond_emb, W, b, ln_g, ln_b):
    nc, _ = _sc_info()
    x9r = jnp.pad(x, ((0, _NP - _N), (0, 0))).T.reshape(9 * (_NP // 40), 40)
    aembs = [atom_emb[f] for f in range(9)]
    bcat = bond_emb.reshape(24, _H)
    epad = _EP - _E
    src = jnp.pad(edge_index[0], (0, epad)).reshape(_ECHUNKS, _CH)
    dst = jnp.pad(edge_index[1], (0, epad),
                  constant_values=_N).reshape(_ECHUNKS, _CH)
    ea0 = jnp.pad(edge_attr[:, 0], (0, epad)).reshape(_ECHUNKS, _CH)
    ea1 = jnp.pad(edge_attr[:, 1], (0, epad)).reshape(_ECHUNKS, _CH)
    ea2 = jnp.pad(edge_attr[:, 2], (0, epad)).reshape(_ECHUNKS, _CH)

    sc_encode = _build_sc_encode()
    sc_deg = _build_sc_deg()
    sc_msg = _build_sc_msg()

    henc, ctab, eidx = sc_encode(x9r, *aembs, bcat, ea0, ea1, ea2)
    deg = sc_deg(dst)
    degf = _EPS * (deg[0] + deg[1]) if nc == 2 else _EPS * deg.sum(axis=0)

    m = sc_msg(henc, src, eidx, dst, ctab)
    h, aux = _build_tc_layer(nc, False, False)(
        henc, m, degf, W[0], b[0:1], ln_g[0:1], ln_b[0:1])

    for l in range(1, _LAYERS):
        m = sc_msg(aux, src, eidx, dst, ctab)
        final = l == _LAYERS - 1
        h, aux = _build_tc_layer(nc, True, final)(
            aux, m, degf, h, W[l], b[l:l + 1], ln_g[l:l + 1], ln_b[l:l + 1])

    return aux[:_N]
